# Initial kernel scaffold; baseline (speedup 1.0000x reference)
import jax
import jax.numpy as jnp
from jax.experimental import pallas as pl


def kernel(x, edge_index, edge_attr, track_id, a, params):
    raise NotImplementedError("write your pallas kernel here")


# speedup vs baseline: 2.2180x; 2.2180x over previous
import functools

import jax
import jax.numpy as jnp
from jax import lax
from jax.experimental import pallas as pl
from jax.experimental.pallas import tpu as pltpu
from jax.experimental.pallas import tpu_sc as plsc

H = 128
N_MP = 5
NC = 2
NS = 16
NW = NC * NS
C = 128
LN_EPS = 1e-5


def _mm(x, w):
    return lax.dot_general(x, w, (((1,), (0,)), ((), ())),
                           preferred_element_type=jnp.float32)


def _ln(x, g, b):
    mu = jnp.mean(x, axis=-1, keepdims=True)
    var = jnp.mean((x - mu) ** 2, axis=-1, keepdims=True)
    return (x - mu) * lax.rsqrt(var + LN_EPS) * g + b


def _wspec(shape):
    nd = len(shape)
    return pl.BlockSpec(shape, lambda i, _nd=nd: (0,) * _nd)


def _node_in_body(nfin, w1, b1, w2, b2, w3, b3, g, be, w1a, w1b, b1e,
                  nf_ref, p_ref, q_ref):
    h = jnp.maximum(_mm(nfin[...], w1[...]) + b1[...], 0.0)
    h = jnp.maximum(_mm(h, w2[...]) + b2[...], 0.0)
    nf = _ln(_mm(h, w3[...]) + b3[...], g[...], be[...])
    nf_ref[...] = nf
    p_ref[...] = _mm(nf, w1a[...]) + b1e[...]
    q_ref[...] = _mm(nf, w1b[...])


def _edge_in_body(ea, w1, b1, w2, b2, w3, b3, g, be, w1c, c_ref):
    h = jnp.maximum(_mm(ea[...], w1[...]) + b1[...], 0.0)
    h = jnp.maximum(_mm(h, w2[...]) + b2[...], 0.0)
    ef = _ln(_mm(h, w3[...]) + b3[...], g[...], be[...])
    c_ref[...] = _mm(ef, w1c[...])


def _edge_body(last, g_in, c_in, w2, b2, w3, b3, g, be, w1c, msg_ref,
               *cnew_ref):
    h1 = jnp.maximum(g_in[...] + c_in[...], 0.0)
    h2 = jnp.maximum(_mm(h1, w2[...]) + b2[...], 0.0)
    msg = _ln(_mm(h2, w3[...]) + b3[...], g[...], be[...])
    msg_ref[...] = msg
    if not last:
        cnew_ref[0][...] = c_in[...] + _mm(msg, w1c[...])


def _node_body(nf, pa0, pa1, va, vb, b1n, w2n, b2n, w3n, b3n, gn, ben,
               wo1, bo1, wo2, bo2, wo3, bo3, w1a, w1b, b1e,
               nfn_ref, out_ref, p_ref, q_ref):
    aggr = pa0[...] + pa1[...]
    u = jnp.maximum(_mm(nf[...], va[...]) + _mm(aggr, vb[...]) + b1n[...], 0.0)
    u = jnp.maximum(_mm(u, w2n[...]) + b2n[...], 0.0)
    upd = _ln(_mm(u, w3n[...]) + b3n[...], gn[...], ben[...])
    nfn = nf[...] + upd
    nfn_ref[...] = nfn
    o = jnp.maximum(_mm(nfn, wo1[...]) + bo1[...], 0.0)
    o = jnp.maximum(_mm(o, wo2[...]) + bo2[...], 0.0)
    out_ref[...] = _mm(o, wo3[...]) + bo3[...]
    p_ref[...] = _mm(nfn, w1a[...]) + b1e[...]
    q_ref[...] = _mm(nfn, w1b[...])


def _targets_body(x_ref, tgt_ref):
    x2 = x_ref[:, 2:3]
    x3 = x_ref[:, 3:4]
    x17 = x_ref[:, 17:18]
    sx = (x_ref[:, 18:19] - x2) / N_MP
    sy = (x_ref[:, 19:20] - x3) / N_MP
    cols = ([x2 + sx * float(i + 1) * x17 for i in range(N_MP)] +
            [x3 + sy * float(i + 1) * x17 for i in range(N_MP)])
    tgt_ref[...] = jnp.concatenate(cols, axis=1)


def _gather_call(p, q, dstw, srcw, epad, k_chunks):
    mesh = plsc.VectorSubcoreMesh(core_axis_name="c", subcore_axis_name="s",
                                  num_cores=NC, num_subcores=NS)

    @functools.partial(
        pl.kernel,
        out_type=jax.ShapeDtypeStruct((epad, H), jnp.float32),
        mesh=mesh,
        scratch_types=[
            pltpu.VMEM((C,), jnp.int32),
            pltpu.VMEM((C,), jnp.int32),
            pltpu.VMEM((C, H), jnp.float32),
            pltpu.SemaphoreType.DMA,
        ],
    )
    def k(p_hbm, q_hbm, d_hbm, s_hbm, g_hbm, idx1, idx2, buf, sem):
        cid = lax.axis_index("c")
        sid = lax.axis_index("s")
        wid = cid * NS + sid

        def body(j, carry):
            pltpu.sync_copy(d_hbm.at[wid, j], idx1)
            pltpu.sync_copy(s_hbm.at[wid, j], idx2)
            pltpu.async_copy(p_hbm.at[idx1], buf, sem).wait()
            pltpu.async_copy(q_hbm.at[idx2], buf, sem, add=True).wait()
            pltpu.sync_copy(buf, g_hbm.at[pl.ds((wid * k_chunks + j) * C, C)])
            return carry

        lax.fori_loop(0, k_chunks, body, 0)

    return k(p, q, dstw, srcw)


def _scatter_call(msg, dstw, zeros, npad, k_chunks):
    mesh = plsc.VectorSubcoreMesh(core_axis_name="c", subcore_axis_name="s",
                                  num_cores=NC, num_subcores=NS)
    rows = npad // NS

    @functools.partial(
        pl.kernel,
        out_type=jax.ShapeDtypeStruct((NC, npad, H), jnp.float32),
        mesh=mesh,
        scratch_types=[
            pltpu.VMEM((C,), jnp.int32),
            pltpu.VMEM((C, H), jnp.float32),
            pltpu.VMEM_SHARED((npad, H), jnp.float32),
        ],
    )
    def k(m_hbm, d_hbm, z_hbm, o_hbm, idxb, buf, acc):
        cid = lax.axis_index("c")
        sid = lax.axis_index("s")
        pltpu.sync_copy(z_hbm.at[pl.ds(sid * rows, rows)],
                        acc.at[pl.ds(sid * rows, rows)])
        plsc.subcore_barrier()
        wid = cid * NS + sid

        def body(j, carry):
            pltpu.sync_copy(m_hbm.at[pl.ds((wid * k_chunks + j) * C, C)], buf)
            pltpu.sync_copy(d_hbm.at[wid, j], idxb)
            pltpu.sync_copy(buf, acc.at[idxb], add=True)
            return carry

        lax.fori_loop(0, k_chunks, body, 0)
        plsc.subcore_barrier()
        pltpu.sync_copy(acc.at[pl.ds(sid * rows, rows)],
                        o_hbm.at[cid, pl.ds(sid * rows, rows)])

    return k(msg, dstw, zeros)


def kernel(x, edge_index, edge_attr, track_id, a, params):
    n = x.shape[0]
    e = edge_index.shape[1]
    npad = -(-(n + 1) // 1280) * 1280
    k_chunks = -(-e // (NW * C))
    epad = NW * k_chunks * C
    bn = 1280
    be = 2048 if epad % 2048 == 0 else C

    f32 = jnp.float32
    pr = params

    def row(v):
        return v.reshape(1, -1).astype(f32)

    w1e = pr["lin_edge"]["W"][0]
    w1a, w1b, w1c = w1e[:H], w1e[H:2 * H], w1e[2 * H:]
    b1e = row(pr["lin_edge"]["b"][0])
    w1n = pr["lin_node"]["W"][0]
    va, vb = w1n[:H], w1n[H:]

    nfin = jnp.concatenate(
        [x[:, 2:5], x[:, 6:9], x[:, 10:14], a[track_id]], axis=1)
    nfin = jnp.pad(nfin, ((0, npad - n), (0, 0))).astype(f32)
    ea = edge_attr.astype(f32)
    ea = jnp.pad(ea, ((0, epad - e), (0, 8 - ea.shape[1])))
    dst = edge_index[1].astype(jnp.int32)
    src = edge_index[0].astype(jnp.int32)
    dstw = jnp.pad(dst, (0, epad - e), constant_values=n).reshape(
        NW, k_chunks, C)
    srcw = jnp.pad(src, (0, epad - e), constant_values=n).reshape(
        NW, k_chunks, C)
    zeros = jnp.zeros((npad, H), f32)

    ni = pr["node_in"]
    nf, p, q = pl.pallas_call(
        _node_in_body,
        grid=(npad // bn,),
        in_specs=[pl.BlockSpec((bn, 16), lambda i: (i, 0))] +
                 [_wspec(s) for s in [(16, H), (1, H), (H, H), (1, H),
                                      (H, H), (1, H), (1, H), (1, H),
                                      (H, H), (H, H), (1, H)]],
        out_specs=[pl.BlockSpec((bn, H), lambda i: (i, 0))] * 3,
        out_shape=[jax.ShapeDtypeStruct((npad, H), f32)] * 3,
    )(nfin, ni["W"][0], row(ni["b"][0]), ni["W"][1], row(ni["b"][1]),
      ni["W"][2], row(ni["b"][2]), row(ni["g"]), row(ni["be"]),
      w1a, w1b, b1e)

    tgt = pl.pallas_call(
        _targets_body,
        grid=(n // 2000 if n % 2000 == 0 else 1,),
        in_specs=[pl.BlockSpec((2000 if n % 2000 == 0 else n, 22),
                               lambda i: (i, 0))],
        out_specs=pl.BlockSpec((2000 if n % 2000 == 0 else n, 2 * N_MP),
                               lambda i: (i, 0)),
        out_shape=jax.ShapeDtypeStruct((n, 2 * N_MP), f32),
    )(x.astype(f32))

    ei = pr["edge_in"]
    w1ep = jnp.pad(ei["W"][0].astype(f32), ((0, 8 - ei["W"][0].shape[0]),
                                            (0, 0)))
    c = pl.pallas_call(
        _edge_in_body,
        grid=(epad // be,),
        in_specs=[pl.BlockSpec((be, 8), lambda i: (i, 0))] +
                 [_wspec(s) for s in [(8, H), (1, H), (H, H), (1, H),
                                      (H, H), (1, H), (1, H), (1, H),
                                      (H, H)]],
        out_specs=pl.BlockSpec((be, H), lambda i: (i, 0)),
        out_shape=jax.ShapeDtypeStruct((epad, H), f32),
    )(ea, w1ep, row(ei["b"][0]), ei["W"][1], row(ei["b"][1]),
      ei["W"][2], row(ei["b"][2]), row(ei["g"]), row(ei["be"]), w1c)

    le = pr["lin_edge"]
    ln = pr["lin_node"]
    no = pr["node_out"]
    wo3 = jnp.pad(no["W"][2].astype(f32), ((0, 0), (0, H - no["W"][2].shape[1])))
    bo3 = jnp.pad(row(no["b"][2]), ((0, 0), (0, H - no["b"][2].shape[0])))

    outs = []
    for t in range(N_MP):
        g_sum = _gather_call(p, q, dstw, srcw, epad, k_chunks)

        last = t == N_MP - 1
        edge_outs = pl.pallas_call(
            functools.partial(_edge_body, last),
            grid=(epad // be,),
            in_specs=[pl.BlockSpec((be, H), lambda i: (i, 0))] * 2 +
                     [_wspec(s) for s in [(H, H), (1, H), (H, H), (1, H),
                                          (1, H), (1, H), (H, H)]],
            out_specs=[pl.BlockSpec((be, H), lambda i: (i, 0))] *
                      (1 if last else 2),
            out_shape=[jax.ShapeDtypeStruct((epad, H), f32)] *
                      (1 if last else 2),
        )(g_sum, c, le["W"][1], row(le["b"][1]), le["W"][2], row(le["b"][2]),
          row(le["g"]), row(le["be"]), w1c)
        if last:
            msg, = edge_outs
        else:
            msg, c = edge_outs

        part = _scatter_call(msg, dstw, zeros, npad, k_chunks)

        nf, out2, p, q = pl.pallas_call(
            _node_body,
            grid=(npad // bn,),
            in_specs=[pl.BlockSpec((bn, H), lambda i: (i, 0))] * 3 +
                     [_wspec(s) for s in [(H, H), (H, H), (1, H), (H, H),
                                          (1, H), (H, H), (1, H), (1, H),
                                          (1, H), (H, H), (1, H), (H, H),
                                          (1, H), (H, H), (1, H), (H, H),
                                          (H, H), (1, H)]],
            out_specs=[pl.BlockSpec((bn, H), lambda i: (i, 0))] * 4,
            out_shape=[jax.ShapeDtypeStruct((npad, H), f32)] * 4,
        )(nf, part[0], part[1], va, vb, row(ln["b"][0]), ln["W"][1],
          row(ln["b"][1]), ln["W"][2], row(ln["b"][2]), row(ln["g"]),
          row(ln["be"]), no["W"][0], row(no["b"][0]), no["W"][1],
          row(no["b"][1]), wo3, bo3, w1a, w1b, b1e)
        outs.append(out2)

    pos = jnp.stack([o[:n, 0] for o in outs] + [o[:n, 1] for o in outs],
                    axis=1)
    return pos, tgt
